# Initial kernel scaffold; baseline (speedup 1.0000x reference)
import jax
import jax.numpy as jnp
from jax.experimental import pallas as pl


def kernel(x, table):
    raise NotImplementedError("write your pallas kernel here")


# speedup vs baseline: 1.0184x; 1.0184x over previous
import functools

import jax
import jax.numpy as jnp
from jax import lax
from jax.experimental import pallas as pl
from jax.experimental.pallas import tpu as pltpu
from jax.experimental.pallas import tpu_sc as plsc

_CH = 128
_K = 4


@functools.lru_cache(maxsize=None)
def _build_gather(B, D):
    info = plsc.get_sparse_core_info()
    NC, NS = info.num_cores, info.num_subcores
    NW = NC * NS
    b_per_w = B // NW
    n_chunks = b_per_w // _CH
    n_groups = n_chunks // _K

    mesh = plsc.VectorSubcoreMesh(core_axis_name="c", subcore_axis_name="s")

    @functools.partial(
        pl.kernel,
        mesh=mesh,
        out_type=jax.ShapeDtypeStruct((B, D), jnp.float32),
        compiler_params=pltpu.CompilerParams(use_tc_tiling_on_sc=False),
        scratch_types=[
            pltpu.VMEM((b_per_w,), jnp.int32),
            pltpu.VMEM((_K, _CH, D), jnp.float32),
            pltpu.SemaphoreType.DMA,
            pltpu.SemaphoreType.DMA,
        ],
    )
    def k(idx_hbm, table_hbm, out_hbm, idx_v, rows_v, gsem, ssem):
        wid = lax.axis_index("s") * NC + lax.axis_index("c")
        base = wid * b_per_w
        pltpu.sync_copy(idx_hbm.at[pl.ds(base, b_per_w)], idx_v)

        def group(g, carry):
            j0 = g * _K
            gathers = [
                pltpu.async_copy(
                    table_hbm.at[idx_v.at[pl.ds((j0 + b) * _CH, _CH)]],
                    rows_v.at[b],
                    gsem,
                )
                for b in range(_K)
            ]
            for d in gathers:
                d.wait()
            stores = [
                pltpu.async_copy(
                    rows_v.at[b],
                    out_hbm.at[pl.ds(base + (j0 + b) * _CH, _CH)],
                    ssem,
                )
                for b in range(_K)
            ]
            for d in stores:
                d.wait()
            return carry

        lax.fori_loop(0, n_groups, group, 0)

    return k


def kernel(x, table):
    B = x.size
    D = table.shape[1]
    idx_flat = x.reshape(-1).astype(jnp.int32)
    out = _build_gather(B, D)(idx_flat, table)
    return out.reshape(*x.shape, D)


# speedup vs baseline: 1.0331x; 1.0144x over previous
import functools

import jax
import jax.numpy as jnp
from jax import lax
from jax.experimental import pallas as pl
from jax.experimental.pallas import tpu as pltpu
from jax.experimental.pallas import tpu_sc as plsc

_CH = 128
_GPS = 4
_SUP = _CH * _GPS


@functools.lru_cache(maxsize=None)
def _build_gather(B, D):
    info = plsc.get_sparse_core_info()
    NC, NS = info.num_cores, info.num_subcores
    NW = NC * NS
    b_per_w = B // NW
    n_sup = b_per_w // _SUP

    mesh = plsc.VectorSubcoreMesh(core_axis_name="c", subcore_axis_name="s")

    @functools.partial(
        pl.kernel,
        mesh=mesh,
        out_type=jax.ShapeDtypeStruct((B, D), jnp.float32),
        compiler_params=pltpu.CompilerParams(use_tc_tiling_on_sc=False),
        scratch_types=[
            pltpu.VMEM((b_per_w,), jnp.int32),
            pltpu.VMEM((_SUP, D), jnp.float32),
            pltpu.VMEM((_SUP, D), jnp.float32),
            pltpu.SemaphoreType.DMA,
            pltpu.SemaphoreType.DMA,
            pltpu.SemaphoreType.DMA,
            pltpu.SemaphoreType.DMA,
        ],
    )
    def k(idx_hbm, table_hbm, out_hbm, idx_v, buf_a, buf_b, gsem_a, gsem_b,
          ssem_a, ssem_b):
        wid = lax.axis_index("s") * NC + lax.axis_index("c")
        base = wid * b_per_w
        pltpu.sync_copy(idx_hbm.at[pl.ds(base, b_per_w)], idx_v)

        bufs = (buf_a, buf_b)
        gsems = (gsem_a, gsem_b)
        ssems = (ssem_a, ssem_b)

        def fire_gathers(g, p):
            for b in range(_GPS):
                off = pl.multiple_of(g * _SUP + b * _CH, _CH)
                pltpu.async_copy(
                    table_hbm.at[idx_v.at[pl.ds(off, _CH)]],
                    bufs[p].at[pl.ds(b * _CH, _CH)],
                    gsems[p],
                )

        def drain_gathers(g, p):
            for b in range(_GPS):
                off = pl.multiple_of(g * _SUP + b * _CH, _CH)
                pltpu.make_async_copy(
                    table_hbm.at[idx_v.at[pl.ds(off, _CH)]],
                    bufs[p].at[pl.ds(b * _CH, _CH)],
                    gsems[p],
                ).wait()

        def fire_store(g, p):
            pltpu.async_copy(
                bufs[p], out_hbm.at[pl.ds(base + g * _SUP, _SUP)], ssems[p]
            )

        def drain_store(g, p):
            pltpu.make_async_copy(
                bufs[p], out_hbm.at[pl.ds(base + g * _SUP, _SUP)], ssems[p]
            ).wait()

        def step(g, p, fire_next):
            drain_gathers(g, p)
            fire_store(g, p)
            if fire_next:
                drain_store(g, p)
                fire_gathers(g + 2, p)

        fire_gathers(0, 0)
        fire_gathers(1, 1)

        n_pairs = (n_sup - 3) // 2

        def pair(i, carry):
            g0 = i * 2
            step(g0, 0, True)
            step(g0 + 1, 1, True)
            return carry

        lax.fori_loop(0, n_pairs, pair, 0)

        for g in range(2 * n_pairs, n_sup):
            step(g, g % 2, g + 2 < n_sup)

        drain_store(n_sup - 2, (n_sup - 2) % 2)
        drain_store(n_sup - 1, (n_sup - 1) % 2)

    return k


def kernel(x, table):
    B = x.size
    D = table.shape[1]
    idx_flat = x.reshape(-1).astype(jnp.int32)
    out = _build_gather(B, D)(idx_flat, table)
    return out.reshape(*x.shape, D)


# speedup vs baseline: 1.2441x; 1.2043x over previous
import functools

import jax
import jax.numpy as jnp
from jax import lax
from jax.experimental import pallas as pl
from jax.experimental.pallas import tpu as pltpu
from jax.experimental.pallas import tpu_sc as plsc

_CH = 128
_SPS = 4
_DP = 128
_BP = 104


@functools.lru_cache(maxsize=None)
def _build_gather(A, Bd):
    info = plsc.get_sparse_core_info()
    NC, NS = info.num_cores, info.num_subcores
    NW = NC * NS
    a_per_w = A // NW
    b_per_w = a_per_w * Bd
    sup_rows = _SPS * Bd
    n_sup = a_per_w // _SPS
    chunks = []
    off = 0
    while off < sup_rows:
        n = min(_CH, sup_rows - off)
        chunks.append((off, n))
        off += n

    mesh = plsc.VectorSubcoreMesh(core_axis_name="c", subcore_axis_name="s")

    @functools.partial(
        pl.kernel,
        mesh=mesh,
        out_type=jax.ShapeDtypeStruct((A, _BP, _DP), jnp.float32),
        compiler_params=pltpu.CompilerParams(use_tc_tiling_on_sc=False),
        scratch_types=[
            pltpu.VMEM((b_per_w,), jnp.int32),
            pltpu.VMEM((sup_rows, 64), jnp.float32),
            pltpu.VMEM((sup_rows, 64), jnp.float32),
            pltpu.SemaphoreType.DMA,
            pltpu.SemaphoreType.DMA,
            pltpu.SemaphoreType.DMA,
            pltpu.SemaphoreType.DMA,
        ],
    )
    def k(idx_hbm, table_hbm, out_hbm, idx_v, buf_a, buf_b, gsem_a, gsem_b,
          ssem_a, ssem_b):
        wid = lax.axis_index("s") * NC + lax.axis_index("c")
        base_a = wid * a_per_w
        pltpu.sync_copy(idx_hbm.at[pl.ds(wid * b_per_w, b_per_w)], idx_v)

        bufs = (buf_a, buf_b)
        gsems = (gsem_a, gsem_b)
        ssems = (ssem_a, ssem_b)

        def fire_gathers(g, p):
            for (o, n) in chunks:
                off = pl.multiple_of(g * sup_rows + o, 8)
                pltpu.async_copy(
                    table_hbm.at[idx_v.at[pl.ds(off, n)]],
                    bufs[p].at[pl.ds(o, n)],
                    gsems[p],
                )

        def drain_gathers(g, p):
            for (o, n) in chunks:
                off = pl.multiple_of(g * sup_rows + o, 8)
                pltpu.make_async_copy(
                    table_hbm.at[idx_v.at[pl.ds(off, n)]],
                    bufs[p].at[pl.ds(o, n)],
                    gsems[p],
                ).wait()

        def fire_stores(g, p):
            for s in range(_SPS):
                a = base_a + g * _SPS + s
                pltpu.async_copy(
                    bufs[p].at[pl.ds(s * Bd, Bd)],
                    out_hbm.at[a, pl.ds(0, Bd), pl.ds(0, 64)],
                    ssems[p],
                )

        def drain_stores(g, p):
            for s in range(_SPS):
                a = base_a + g * _SPS + s
                pltpu.make_async_copy(
                    bufs[p].at[pl.ds(s * Bd, Bd)],
                    out_hbm.at[a, pl.ds(0, Bd), pl.ds(0, 64)],
                    ssems[p],
                ).wait()

        def step(g, p, fire_next):
            drain_gathers(g, p)
            fire_stores(g, p)
            if fire_next:
                drain_stores(g, p)
                fire_gathers(g + 2, p)

        fire_gathers(0, 0)
        fire_gathers(1, 1)

        n_pairs = (n_sup - 3) // 2

        def pair(i, carry):
            g0 = i * 2
            step(g0, 0, True)
            step(g0 + 1, 1, True)
            return carry

        lax.fori_loop(0, n_pairs, pair, 0)

        for g in range(2 * n_pairs, n_sup):
            step(g, g % 2, g + 2 < n_sup)

        drain_stores(n_sup - 2, (n_sup - 2) % 2)
        drain_stores(n_sup - 1, (n_sup - 1) % 2)

    return k


def kernel(x, table):
    A, Bd = x.shape
    D = table.shape[1]
    idx_flat = x.reshape(-1).astype(jnp.int32)
    out_p = _build_gather(A, Bd)(idx_flat, table)
    return out_p[:, :Bd, :D]


# speedup vs baseline: 2.0632x; 1.6584x over previous
import functools

import jax
import jax.numpy as jnp
from jax import lax
from jax.experimental import pallas as pl
from jax.experimental.pallas import tpu as pltpu
from jax.experimental.pallas import tpu_sc as plsc

_CH = 128
_SPS = 4
_DP = 128

@functools.lru_cache(maxsize=None)
def _build_gather(A, Bd, D):
    bp = -(-Bd // 8) * 8
    info = plsc.get_sparse_core_info()
    NC, NS = info.num_cores, info.num_subcores
    NW = NC * NS
    a_per_w = A // NW
    b_per_w = a_per_w * Bd
    sup_rows = _SPS * Bd
    n_sup = a_per_w // _SPS
    chunks = []
    off = 0
    while off < sup_rows:
        n = min(_CH, sup_rows - off)
        chunks.append((off, n))
        off += n

    mesh = plsc.VectorSubcoreMesh(core_axis_name="c", subcore_axis_name="s")

    @functools.partial(
        pl.kernel,
        mesh=mesh,
        out_type=jax.ShapeDtypeStruct((A, bp, _DP), jnp.float32),
        compiler_params=pltpu.CompilerParams(use_tc_tiling_on_sc=False),
        scratch_types=[
            pltpu.VMEM((b_per_w,), jnp.int32),
            pltpu.VMEM((sup_rows, _DP), jnp.float32),
            pltpu.VMEM((sup_rows, _DP), jnp.float32),
            pltpu.SemaphoreType.DMA,
            pltpu.SemaphoreType.DMA,
            pltpu.SemaphoreType.DMA,
            pltpu.SemaphoreType.DMA,
        ],
    )
    def k(idx_hbm, table_hbm, out_hbm, idx_v, buf_a, buf_b, gsem_a, gsem_b,
          ssem_a, ssem_b):
        wid = lax.axis_index("s") * NC + lax.axis_index("c")
        base_a = wid * a_per_w
        pltpu.sync_copy(idx_hbm.at[pl.ds(wid * b_per_w, b_per_w)], idx_v)

        bufs = (buf_a, buf_b)
        gsems = (gsem_a, gsem_b)
        ssems = (ssem_a, ssem_b)

        def fire_gathers(g, p):
            for (o, n) in chunks:
                off = pl.multiple_of(g * sup_rows + o, 8)
                pltpu.async_copy(
                    table_hbm.at[idx_v.at[pl.ds(off, n)]],
                    bufs[p].at[pl.ds(o, n)],
                    gsems[p],
                )

        def drain_gathers(g, p):
            for (o, n) in chunks:
                off = pl.multiple_of(g * sup_rows + o, 8)
                pltpu.make_async_copy(
                    table_hbm.at[idx_v.at[pl.ds(off, n)]],
                    bufs[p].at[pl.ds(o, n)],
                    gsems[p],
                ).wait()

        def fire_stores(g, p):
            for s in range(_SPS):
                a = base_a + g * _SPS + s
                pltpu.async_copy(
                    bufs[p].at[pl.ds(s * Bd, Bd), pl.ds(0, D)],
                    out_hbm.at[a, pl.ds(0, Bd), pl.ds(0, D)],
                    ssems[p],
                )

        def drain_stores(g, p):
            for s in range(_SPS):
                a = base_a + g * _SPS + s
                pltpu.make_async_copy(
                    bufs[p].at[pl.ds(s * Bd, Bd), pl.ds(0, D)],
                    out_hbm.at[a, pl.ds(0, Bd), pl.ds(0, D)],
                    ssems[p],
                ).wait()

        def step(g, p, fire_next):
            drain_gathers(g, p)
            fire_stores(g, p)
            if fire_next:
                drain_stores(g, p)
                fire_gathers(g + 2, p)

        fire_gathers(0, 0)
        fire_gathers(1, 1)

        n_pairs = (n_sup - 3) // 2

        def pair(i, carry):
            g0 = i * 2
            step(g0, 0, True)
            step(g0 + 1, 1, True)
            return carry

        lax.fori_loop(0, n_pairs, pair, 0)

        for g in range(2 * n_pairs, n_sup):
            step(g, g % 2, g + 2 < n_sup)

        drain_stores(n_sup - 2, (n_sup - 2) % 2)
        drain_stores(n_sup - 1, (n_sup - 1) % 2)

    return k


_TBLK = 16384


@functools.lru_cache(maxsize=None)
def _build_row_major(V, D):
    def body(t_ref, out_ref):
        out_ref[:, :D] = jnp.swapaxes(t_ref[...], 0, 1)

    grid = (V + _TBLK - 1) // _TBLK
    return pl.pallas_call(
        body,
        grid=(grid,),
        in_specs=[pl.BlockSpec((D, _TBLK), lambda i: (0, i))],
        out_specs=pl.BlockSpec((_TBLK, _DP), lambda i: (i, 0)),
        out_shape=jax.ShapeDtypeStruct((V, _DP), jnp.float32),
    )


def kernel(x, table):
    A, Bd = x.shape
    V, D = table.shape
    idx_flat = x.reshape(-1).astype(jnp.int32)
    table_p = _build_row_major(V, D)(table.T)
    out_p = _build_gather(A, Bd, D)(idx_flat, table_p)
    return out_p[:, :Bd, :D]
